# baseline (speedup 1.0000x reference)
import functools

import jax
import jax.numpy as jnp
from jax import lax
from jax.experimental import pallas as pl
from jax.experimental.pallas import tpu as pltpu
from jax.experimental.pallas import tpu_sc as plsc

_info = plsc.get_sparse_core_info()
_NC, _NS = _info.num_cores, _info.num_subcores
_NW = _NC * _NS

_DP = 128
_BW = 128
_NRING = 3
_L = 16


def _sc_gather(table_padded, idx_flat, b, t, d):
    toks_per_w = _BW * t
    mesh = plsc.VectorSubcoreMesh(core_axis_name="c", subcore_axis_name="s")

    @functools.partial(
        pl.kernel,
        mesh=mesh,
        compiler_params=pltpu.CompilerParams(needs_layout_passes=False),
        out_type=jax.ShapeDtypeStruct((t, d // 8, _NW, 8, _BW), jnp.float32),
        scratch_types=[
            pltpu.VMEM((toks_per_w,), jnp.int32),
            pltpu.VMEM((t, _BW), jnp.int32),
            pltpu.VMEM((_NRING, _BW, _DP), jnp.float32),
            pltpu.VMEM((2, d // 8, 8, _BW), jnp.float32),
            pltpu.SemaphoreType.DMA((_NRING,)),
            pltpu.SemaphoreType.DMA((2,)),
        ],
    )
    def k(table_hbm, idx_hbm, out_hbm, idx_v, idxt_v, g_v, tr_v, gsem, ssem):
        wid = lax.axis_index("s") * _NC + lax.axis_index("c")
        base = wid * toks_per_w
        pltpu.sync_copy(idx_hbm.at[pl.ds(base, toks_per_w)], idx_v)

        lanes = lax.iota(jnp.int32, _L)

        def idx_body(tt, carry):
            for kk in range(_BW // _L):
                off = lanes * t + (kk * _L * t) + tt
                v = plsc.load_gather(idx_v, [off])
                idxt_v[tt, pl.ds(kk * _L, _L)] = v
            return carry

        lax.fori_loop(0, t, idx_body, 0)

        def gather(tt, p):
            return pltpu.make_async_copy(
                table_hbm.at[idxt_v.at[tt]], g_v.at[p], gsem.at[p])

        def store(tt, q):
            return pltpu.make_async_copy(
                tr_v.at[q], out_hbm.at[tt, pl.ds(0, d // 8), wid], ssem.at[q])

        def transpose(p, q):
            gp = g_v.at[p]

            def tr_body(f, carry):
                fg = f // 8
                fr = f - fg * 8
                col = jnp.zeros((_L,), jnp.int32) + f
                for kk in range(_BW // _L):
                    v = plsc.load_gather(gp, [lanes + kk * _L, col])
                    tr_v[q, fg, fr, pl.ds(kk * _L, _L)] = v
                return carry

            lax.fori_loop(0, d, tr_body, 0)

        for p in range(_NRING):
            gather(p, p).start()

        def body(tt, p, q):
            gather(tt, p).wait()

            @pl.when(tt >= 2)
            def _():
                store(tt, q).wait()

            transpose(p, q)
            store(tt, q).start()

            @pl.when(tt + _NRING < t)
            def _():
                gather(tt + _NRING, p).start()

        nper = 2 * _NRING
        nfull = (t // nper) * nper

        def outer(gg, carry):
            for j in range(nper):
                body(gg * nper + j, j % _NRING, j % 2)
            return carry

        lax.fori_loop(0, t // nper, outer, 0)

        for tt in range(nfull, t):
            body(tt, tt % _NRING, tt % 2)

        for tt in range(t - 2, t):
            store(tt, tt % 2).wait()

    return k(table_padded, idx_flat)


def kernel(token_ids, embedding_matrix):
    b, t = token_ids.shape
    d = embedding_matrix.shape[1]
    table_padded = jnp.pad(embedding_matrix, ((0, 0), (0, _DP - d)))
    idx_flat = token_ids.astype(jnp.int32).reshape(-1)
    out5 = _sc_gather(table_padded, idx_flat, b, t, d)
    return out5.transpose(2, 4, 0, 1, 3).reshape(b, t, d)


# speedup vs baseline: 1.7499x; 1.7499x over previous
import functools

import jax
import jax.numpy as jnp
from jax import lax
from jax.experimental import pallas as pl
from jax.experimental.pallas import tpu as pltpu
from jax.experimental.pallas import tpu_sc as plsc

_info = plsc.get_sparse_core_info()
_NC, _NS = _info.num_cores, _info.num_subcores
_NW = _NC * _NS

_DP = 128
_CH = 128
_NRING = 5
_LEAD = 3


def _sc_gather(table_padded, idx_flat):
    n = idx_flat.shape[0]
    toks_per_w = n // _NW
    nchunks = toks_per_w // _CH
    mesh = plsc.VectorSubcoreMesh(core_axis_name="c", subcore_axis_name="s")

    @functools.partial(
        pl.kernel,
        mesh=mesh,
        out_type=jax.ShapeDtypeStruct((n, _DP), jnp.float32),
        scratch_types=[
            pltpu.VMEM((toks_per_w,), jnp.int32),
            pltpu.VMEM((_NRING, _CH, _DP), jnp.float32),
            pltpu.SemaphoreType.DMA((_NRING,)),
            pltpu.SemaphoreType.DMA((_NRING,)),
        ],
    )
    def k(table_hbm, idx_hbm, out_hbm, idx_v, rows_v, gsem, ssem):
        wid = lax.axis_index("s") * _NC + lax.axis_index("c")
        base = wid * toks_per_w
        pltpu.sync_copy(idx_hbm.at[pl.ds(base, toks_per_w)], idx_v)

        def gather(c, p):
            return pltpu.make_async_copy(
                table_hbm.at[idx_v.at[pl.ds(c * _CH, _CH)]],
                rows_v.at[p], gsem.at[p])

        def store(c, p):
            return pltpu.make_async_copy(
                rows_v.at[p], out_hbm.at[pl.ds(base + c * _CH, _CH)],
                ssem.at[p])

        for p in range(_LEAD):
            gather(p, p).start()

        def body(c, p):
            gather(c, p).wait()
            store(c, p).start()
            np_ = (p + _LEAD) % _NRING

            @pl.when(c >= _NRING - _LEAD)
            def _():
                store(c - (_NRING - _LEAD), np_).wait()

            @pl.when(c + _LEAD < nchunks)
            def _():
                gather(c + _LEAD, np_).start()

        def outer(gg, carry):
            for j in range(_NRING):
                body(gg * _NRING + j, j)
            return carry

        lax.fori_loop(0, nchunks // _NRING, outer, 0)

        for c in range((nchunks // _NRING) * _NRING, nchunks):
            body(c, c % _NRING)

        for c in range(nchunks - (_NRING - _LEAD), nchunks):
            store(c, c % _NRING).wait()

    return k(table_padded, idx_flat)


def kernel(token_ids, embedding_matrix):
    b, t = token_ids.shape
    d = embedding_matrix.shape[1]
    table_padded = jnp.pad(embedding_matrix, ((0, 0), (0, _DP - d)))
    idx_flat = token_ids.astype(jnp.int32).reshape(-1)
    out_padded = _sc_gather(table_padded, idx_flat)
    return out_padded.reshape(b, t, _DP)[:, :, :d]
